# Initial kernel scaffold; baseline (speedup 1.0000x reference)
import jax
import jax.numpy as jnp
from jax.experimental import pallas as pl


def kernel(input, weight):
    raise NotImplementedError("write your pallas kernel here")


# speedup vs baseline: 4.1030x; 4.1030x over previous
import functools

import jax
import jax.numpy as jnp
from jax import lax
from jax.experimental import pallas as pl
from jax.experimental.pallas import tpu as pltpu
from jax.experimental.pallas import tpu_sc as plsc

D = 32


@functools.partial(jax.jit, static_argnums=())
def _gather_sc(flat_idx, weight):
    B = flat_idx.shape[0]
    info = plsc.get_sparse_core_info()
    NC, NS = info.num_cores, info.num_subcores
    NW = NC * NS
    b_per_w = B // NW
    CHUNK = 128
    n_chunks = b_per_w // CHUNK

    mesh = plsc.VectorSubcoreMesh(core_axis_name="c", subcore_axis_name="s")

    @functools.partial(
        pl.kernel,
        mesh=mesh,
        compiler_params=pltpu.CompilerParams(use_tc_tiling_on_sc=False),
        out_type=jax.ShapeDtypeStruct((B, D), jnp.float32),
        scratch_types=[
            pltpu.VMEM((b_per_w,), jnp.int32),
            pltpu.VMEM((CHUNK, D), jnp.float32),
            pltpu.SemaphoreType.DMA,
        ],
    )
    def k(idx_hbm, table_hbm, out_hbm, idx_v, rows_v, sem):
        wid = lax.axis_index("s") * NC + lax.axis_index("c")
        base = wid * b_per_w
        pltpu.sync_copy(idx_hbm.at[pl.ds(base, b_per_w)], idx_v)

        def body(j, carry):
            off = pl.multiple_of(j * CHUNK, 8)
            pltpu.async_copy(
                table_hbm.at[idx_v.at[pl.ds(off, CHUNK)]], rows_v, sem
            ).wait()
            pltpu.sync_copy(rows_v, out_hbm.at[pl.ds(base + off, CHUNK)])
            return carry

        lax.fori_loop(0, n_chunks, body, 0)

    return k(flat_idx, weight)


def kernel(input, weight):
    B = input.shape[0] * input.shape[1]
    flat = input.reshape(B)
    out = _gather_sc(flat, weight)
    return out.reshape(input.shape[0], input.shape[1], D)


# speedup vs baseline: 4.3920x; 1.0705x over previous
import functools

import jax
import jax.numpy as jnp
from jax import lax
from jax.experimental import pallas as pl
from jax.experimental.pallas import tpu as pltpu
from jax.experimental.pallas import tpu_sc as plsc

D = 32
CHUNK = 128
NBUF = 4
K = 2


@jax.jit
def _gather_sc(flat_idx, weight):
    B = flat_idx.shape[0]
    info = plsc.get_sparse_core_info()
    NC, NS = info.num_cores, info.num_subcores
    NW = NC * NS
    b_per_w = B // NW
    n_chunks = b_per_w // CHUNK
    assert n_chunks % NBUF == 0 and NBUF > K

    mesh = plsc.VectorSubcoreMesh(core_axis_name="c", subcore_axis_name="s")

    @functools.partial(
        pl.kernel,
        mesh=mesh,
        compiler_params=pltpu.CompilerParams(use_tc_tiling_on_sc=False),
        out_type=jax.ShapeDtypeStruct((B, D), jnp.float32),
        scratch_types=[
            pltpu.VMEM((b_per_w,), jnp.int32),
            pltpu.VMEM((NBUF, CHUNK, D), jnp.float32),
            pltpu.SemaphoreType.DMA((NBUF,)),
            pltpu.SemaphoreType.DMA((NBUF,)),
        ],
    )
    def k(idx_hbm, table_hbm, out_hbm, idx_v, rows_v, gsem, ssem):
        wid = lax.axis_index("s") * NC + lax.axis_index("c")
        base = wid * b_per_w
        pltpu.sync_copy(idx_hbm.at[pl.ds(base, b_per_w)], idx_v)

        def start_gather(j, b):
            off = pl.multiple_of(j * CHUNK, 8)
            pltpu.async_copy(
                table_hbm.at[idx_v.at[pl.ds(off, CHUNK)]],
                rows_v.at[b],
                gsem.at[b],
            )

        def drain_gather(b):
            pltpu.make_async_copy(
                table_hbm.at[idx_v.at[pl.ds(0, CHUNK)]],
                rows_v.at[b],
                gsem.at[b],
            ).wait()

        def start_scatter(j, b):
            off = pl.multiple_of(j * CHUNK, 8)
            pltpu.async_copy(
                rows_v.at[b],
                out_hbm.at[pl.ds(base + off, CHUNK)],
                ssem.at[b],
            )

        def drain_scatter(b):
            pltpu.make_async_copy(
                rows_v.at[b],
                out_hbm.at[pl.ds(base, CHUNK)],
                ssem.at[b],
            ).wait()

        for b in range(K):
            start_gather(b, b)

        @pl.loop(0, n_chunks, step=NBUF)
        def _(j0):
            for b in range(NBUF):
                j = j0 + b
                jn = j + K
                bn = (b + K) % NBUF

                @pl.when(jn < n_chunks)
                def _():
                    @pl.when(jn >= NBUF)
                    def _():
                        drain_scatter(bn)

                    start_gather(jn, bn)

                drain_gather(b)
                start_scatter(j, b)

        for b in range(NBUF):
            drain_scatter(b)

    return k(flat_idx, weight)


def kernel(input, weight):
    B = input.shape[0] * input.shape[1]
    flat = input.reshape(B)
    out = _gather_sc(flat, weight)
    return out.reshape(input.shape[0], input.shape[1], D)


# speedup vs baseline: 4.4154x; 1.0053x over previous
import functools

import jax
import jax.numpy as jnp
from jax import lax
from jax.experimental import pallas as pl
from jax.experimental.pallas import tpu as pltpu
from jax.experimental.pallas import tpu_sc as plsc

D = 32
CHUNK = 512
NBUF = 5
K = 2


@jax.jit
def _gather_sc(flat_idx, weight):
    B = flat_idx.shape[0]
    info = plsc.get_sparse_core_info()
    NC, NS = info.num_cores, info.num_subcores
    NW = NC * NS
    b_per_w = B // NW
    n_chunks = b_per_w // CHUNK
    assert n_chunks % NBUF == 0 and NBUF > K

    mesh = plsc.VectorSubcoreMesh(core_axis_name="c", subcore_axis_name="s")

    @functools.partial(
        pl.kernel,
        mesh=mesh,
        compiler_params=pltpu.CompilerParams(use_tc_tiling_on_sc=False),
        out_type=jax.ShapeDtypeStruct((B, D), jnp.float32),
        scratch_types=[
            pltpu.VMEM((b_per_w,), jnp.int32),
            pltpu.VMEM((NBUF, CHUNK, D), jnp.float32),
            pltpu.SemaphoreType.DMA((NBUF,)),
            pltpu.SemaphoreType.DMA((NBUF,)),
        ],
    )
    def k(idx_hbm, table_hbm, out_hbm, idx_v, rows_v, gsem, ssem):
        wid = lax.axis_index("s") * NC + lax.axis_index("c")
        base = wid * b_per_w
        pltpu.sync_copy(idx_hbm.at[pl.ds(base, b_per_w)], idx_v)

        def start_gather(j, b):
            off = pl.multiple_of(j * CHUNK, 8)
            pltpu.async_copy(
                table_hbm.at[idx_v.at[pl.ds(off, CHUNK)]],
                rows_v.at[b],
                gsem.at[b],
            )

        def drain_gather(b):
            pltpu.make_async_copy(
                table_hbm.at[idx_v.at[pl.ds(0, CHUNK)]],
                rows_v.at[b],
                gsem.at[b],
            ).wait()

        def start_scatter(j, b):
            off = pl.multiple_of(j * CHUNK, 8)
            pltpu.async_copy(
                rows_v.at[b],
                out_hbm.at[pl.ds(base + off, CHUNK)],
                ssem.at[b],
            )

        def drain_scatter(b):
            pltpu.make_async_copy(
                rows_v.at[b],
                out_hbm.at[pl.ds(base, CHUNK)],
                ssem.at[b],
            ).wait()

        for b in range(K):
            start_gather(b, b)

        @pl.loop(0, n_chunks, step=NBUF)
        def _(j0):
            for b in range(NBUF):
                j = j0 + b
                jn = j + K
                bn = (b + K) % NBUF

                @pl.when(jn < n_chunks)
                def _():
                    @pl.when(jn >= NBUF)
                    def _():
                        drain_scatter(bn)

                    start_gather(jn, bn)

                drain_gather(b)
                start_scatter(j, b)

        for b in range(NBUF):
            drain_scatter(b)

    return k(flat_idx, weight)


def kernel(input, weight):
    B = input.shape[0] * input.shape[1]
    flat = input.reshape(B)
    out = _gather_sc(flat, weight)
    return out.reshape(input.shape[0], input.shape[1], D)
